# baseline (speedup 1.0000x reference)
import functools

import jax
import jax.numpy as jnp
from jax import lax
from jax.experimental import pallas as pl
from jax.experimental.pallas import tpu as pltpu
from jax.experimental.pallas import tpu_sc as plsc

B, T = 1024, 512
HD_T, HD_L = 768, 128
NCOORD = 6
MAX_COORD = 1001
NC, NS = 2, 16
NW = NC * NS
BPW = B // NW
T_BLK = 16
NTB = T // T_BLK
LROWS = T_BLK * NCOORD
L = 16
NBUF = 4
NQ = BPW // NBUF

_mesh = plsc.VectorSubcoreMesh(
    core_axis_name="c", subcore_axis_name="s", num_cores=NC, num_subcores=NS
)


@functools.partial(
    pl.kernel,
    out_type=(
        jax.ShapeDtypeStruct((B, T, HD_T), jnp.float32),
        jax.ShapeDtypeStruct((B, T, HD_T), jnp.float32),
    ),
    mesh=_mesh,
    scratch_types=[
        pltpu.VMEM((BPW * T_BLK,), jnp.int32),
        pltpu.VMEM((BPW * LROWS,), jnp.int32),
        pltpu.VMEM((LROWS,), jnp.int32),
        pltpu.VMEM((NBUF, T_BLK, HD_T), jnp.float32),
        pltpu.VMEM((NBUF, LROWS, HD_L), jnp.float32),
        pltpu.VMEM((T_BLK, HD_T), jnp.float32),
        pltpu.VMEM((LROWS, HD_L), jnp.float32),
        pltpu.SemaphoreType.DMA((NBUF,)),
        pltpu.SemaphoreType.DMA((NBUF,)),
        pltpu.SemaphoreType.DMA((NBUF,)),
        pltpu.SemaphoreType.DMA((NBUF,)),
        pltpu.SemaphoreType.DMA,
        pltpu.SemaphoreType.DMA,
    ],
)
def _emb_kernel(words_hbm, bbox_hbm, offp_hbm, lang_hbm, comb_hbm,
                post_hbm, posl_hbm, out_l, out_t,
                widx, lidx, offp, tbufs, lbufs, post, posl,
                sem_gt, sem_gl, sem_st, sem_sl, sem_pp, sem_pl):
    wid = lax.axis_index("s") * NC + lax.axis_index("c")
    pltpu.sync_copy(offp_hbm, offp)
    wbase = wid * (BPW * T_BLK)
    bbase = wid * (BPW * LROWS)

    def issue_g(p, i):
        pltpu.async_copy(
            lang_hbm.at[widx.at[pl.ds(i * T_BLK, T_BLK)]],
            tbufs.at[p], sem_gt.at[p],
        )
        pltpu.async_copy(
            comb_hbm.at[lidx.at[pl.ds(i * LROWS, LROWS)]],
            lbufs.at[p], sem_gl.at[p],
        )

    def wait_g(p):
        pltpu.make_async_copy(
            lang_hbm.at[widx.at[pl.ds(0, T_BLK)]], tbufs.at[p], sem_gt.at[p]
        ).wait()
        pltpu.make_async_copy(
            comb_hbm.at[lidx.at[pl.ds(0, LROWS)]], lbufs.at[p], sem_gl.at[p]
        ).wait()

    def wait_s(p):
        pltpu.make_async_copy(
            tbufs.at[p], out_t.at[0, pl.ds(0, T_BLK), :], sem_st.at[p]
        ).wait()

        @pl.loop(0, NCOORD)
        def _wait_sl(c):
            pltpu.make_async_copy(
                lbufs.at[p, pl.ds(0, T_BLK), :],
                out_l.at[0, pl.ds(0, T_BLK), pl.ds(0, HD_L)],
                sem_sl.at[p],
            ).wait()

    @pl.loop(0, NTB)
    def _tb_loop(tb):
        t0 = tb * T_BLK
        cp_pp = pltpu.async_copy(
            post_hbm.at[pl.ds(t0, T_BLK), :], post, sem_pp
        )
        cp_pl = pltpu.async_copy(
            posl_hbm.at[pl.ds(t0 * NCOORD, LROWS), :], posl, sem_pl
        )
        pltpu.sync_copy(
            words_hbm.at[pl.ds(tb * (B * T_BLK) + wbase, BPW * T_BLK)], widx
        )
        pltpu.sync_copy(
            bbox_hbm.at[pl.ds(tb * (B * LROWS) + bbase, BPW * LROWS)], lidx
        )

        @pl.loop(0, BPW)
        def _mk_idx(g):
            for k in range(NCOORD):
                o = g * LROWS + k * L
                lidx[pl.ds(o, L)] = lidx[pl.ds(o, L)] + offp[pl.ds(k * L, L)]

        issue_g(0, 0)
        issue_g(1, 1)
        cp_pp.wait()
        cp_pl.wait()

        @pl.loop(0, NQ)
        def _q_loop(qq):
            for p in range(NBUF):
                i = qq * NBUF + p
                b = wid * BPW + i
                wait_g(p)

                @pl.loop(0, T_BLK)
                def _add_tl(r):
                    for c2 in range(HD_T // L):
                        plsc.addupdate(
                            tbufs.at[p, r, pl.ds(c2 * L, L)],
                            post[r, pl.ds(c2 * L, L)],
                        )
                        plsc.addupdate(
                            lbufs.at[p, (c2 // 8) * L + r, pl.ds((c2 % 8) * L, L)],
                            posl[(c2 // 8) * L + r, pl.ds((c2 % 8) * L, L)],
                        )

                pltpu.async_copy(
                    tbufs.at[p], out_t.at[b, pl.ds(t0, T_BLK), :], sem_st.at[p]
                )

                @pl.loop(0, NCOORD)
                def _st_l(c):
                    ro = pl.multiple_of(c * T_BLK, T_BLK)
                    co = pl.multiple_of(c * HD_L, HD_L)
                    pltpu.async_copy(
                        lbufs.at[p, pl.ds(ro, T_BLK), :],
                        out_l.at[b, pl.ds(t0, T_BLK), pl.ds(co, HD_L)],
                        sem_sl.at[p],
                    )

                p2 = (p + 2) % NBUF
                if p < 2:
                    @pl.when(qq > 0)
                    def _drain_early():
                        wait_s(p2)

                    issue_g(p2, qq * NBUF + p + 2)
                else:
                    wait_s(p2)

                    @pl.when(qq < NQ - 1)
                    def _refill_late():
                        issue_g(p2, qq * NBUF + p + 2)

        wait_s(2)
        wait_s(3)


def kernel(tokenized_words, tokenized_bbox, lang_table, tlx_table, tly_table,
           brx_table, bry_table, w_table, h_table, box_pos_table,
           text_pos_table):
    words = (
        tokenized_words.astype(jnp.int32)
        .reshape(B, NTB, T_BLK).transpose(1, 0, 2).reshape(-1)
    )
    bbox = (
        tokenized_bbox.astype(jnp.int32)
        .reshape(B, NTB, T_BLK, NCOORD).transpose(1, 0, 3, 2).reshape(-1)
    )
    offp = jnp.repeat(
        jnp.arange(NCOORD, dtype=jnp.int32) * MAX_COORD, T_BLK
    )
    comb = jnp.concatenate(
        [tlx_table, tly_table, brx_table, bry_table, w_table, h_table], axis=0
    )
    post = text_pos_table[:T]
    posl = (
        box_pos_table[:T]
        .reshape(NTB, T_BLK, NCOORD, HD_L).transpose(0, 2, 1, 3)
        .reshape(T * NCOORD, HD_L)
    )
    out_l, out_t = _emb_kernel(words, bbox, offp, lang_table, comb, post, posl)
    return out_l, out_t


# speedup vs baseline: 1.8907x; 1.8907x over previous
import functools

import jax
import jax.numpy as jnp
from jax import lax
from jax.experimental import pallas as pl
from jax.experimental.pallas import tpu as pltpu
from jax.experimental.pallas import tpu_sc as plsc

B, T = 1024, 512
HD_T, HD_L = 768, 128
NCOORD = 6
MAX_COORD = 1001
NC, NS = 2, 16
NW = NC * NS
BPW = B // NW
T_BLK = 16
NTB = T // T_BLK
LROWS = T_BLK * NCOORD
L = 16
NBUF = 4
NQ = BPW // NBUF

_mesh = plsc.VectorSubcoreMesh(
    core_axis_name="c", subcore_axis_name="s", num_cores=NC, num_subcores=NS
)


@functools.partial(
    pl.kernel,
    out_type=(
        jax.ShapeDtypeStruct((B, T, HD_T), jnp.float32),
        jax.ShapeDtypeStruct((B, T, HD_T), jnp.float32),
    ),
    mesh=_mesh,
    scratch_types=[
        pltpu.VMEM((BPW * T_BLK,), jnp.int32),
        pltpu.VMEM((BPW * LROWS,), jnp.int32),
        pltpu.VMEM((LROWS,), jnp.int32),
        pltpu.VMEM((NBUF, T_BLK, HD_T), jnp.float32),
        pltpu.VMEM((NBUF, LROWS, HD_L), jnp.float32),
        pltpu.VMEM((T_BLK, HD_T), jnp.float32),
        pltpu.VMEM((LROWS, HD_L), jnp.float32),
        pltpu.SemaphoreType.DMA((NBUF,)),
        pltpu.SemaphoreType.DMA((NBUF,)),
        pltpu.SemaphoreType.DMA((NBUF,)),
        pltpu.SemaphoreType.DMA((NBUF,)),
        pltpu.SemaphoreType.DMA,
        pltpu.SemaphoreType.DMA,
    ],
)
def _emb_kernel(words_hbm, bbox_hbm, offp_hbm, lang_hbm, comb_hbm,
                post_hbm, posl_hbm, out_l, out_t,
                widx, lidx, offp, tbufs, lbufs, post, posl,
                sem_gt, sem_gl, sem_st, sem_sl, sem_pp, sem_pl):
    wid = lax.axis_index("s") * NC + lax.axis_index("c")
    pltpu.sync_copy(offp_hbm, offp)
    wbase = wid * (BPW * T_BLK)
    bbase = wid * (BPW * LROWS)

    def issue_g(p, i):
        pltpu.async_copy(
            lang_hbm.at[widx.at[pl.ds(i * T_BLK, T_BLK)]],
            tbufs.at[p], sem_gt.at[p],
        )
        pltpu.async_copy(
            comb_hbm.at[lidx.at[pl.ds(i * LROWS, LROWS)]],
            lbufs.at[p], sem_gl.at[p],
        )

    def wait_g(p):
        pltpu.make_async_copy(
            lang_hbm.at[widx.at[pl.ds(0, T_BLK)]], tbufs.at[p], sem_gt.at[p]
        ).wait()
        pltpu.make_async_copy(
            comb_hbm.at[lidx.at[pl.ds(0, LROWS)]], lbufs.at[p], sem_gl.at[p]
        ).wait()

    def wait_s(p):
        pltpu.make_async_copy(
            tbufs.at[p], out_t.at[0, pl.ds(0, T_BLK), :], sem_st.at[p]
        ).wait()

        pltpu.make_async_copy(
            lbufs.at[p], out_l.at[0, pl.ds(0, T_BLK), :], sem_sl.at[p]
        ).wait()

    @pl.loop(0, NTB)
    def _tb_loop(tb):
        t0 = tb * T_BLK
        cp_pp = pltpu.async_copy(
            post_hbm.at[pl.ds(t0, T_BLK), :], post, sem_pp
        )
        cp_pl = pltpu.async_copy(
            posl_hbm.at[pl.ds(t0 * NCOORD, LROWS), :], posl, sem_pl
        )
        pltpu.sync_copy(
            words_hbm.at[pl.ds(tb * (B * T_BLK) + wbase, BPW * T_BLK)], widx
        )
        pltpu.sync_copy(
            bbox_hbm.at[pl.ds(tb * (B * LROWS) + bbase, BPW * LROWS)], lidx
        )

        @pl.loop(0, BPW)
        def _mk_idx(g):
            for k in range(NCOORD):
                o = g * LROWS + k * L
                lidx[pl.ds(o, L)] = lidx[pl.ds(o, L)] + offp[pl.ds(k * L, L)]

        issue_g(0, 0)
        issue_g(1, 1)
        cp_pp.wait()
        cp_pl.wait()

        @pl.loop(0, NQ)
        def _q_loop(qq):
            for p in range(NBUF):
                i = qq * NBUF + p
                b = wid * BPW + i
                wait_g(p)

                @pl.loop(0, T_BLK)
                def _add_t(r):
                    for c2 in range(HD_T // L):
                        plsc.addupdate(
                            tbufs.at[p, r, pl.ds(c2 * L, L)],
                            post[r, pl.ds(c2 * L, L)],
                        )

                pltpu.async_copy(
                    tbufs.at[p], out_t.at[b, pl.ds(t0, T_BLK), :], sem_st.at[p]
                )

                @pl.loop(0, LROWS // 8)
                def _add_l(rr):
                    for j in range(8):
                        r = rr * 8 + j
                        for c2 in range(HD_L // L):
                            plsc.addupdate(
                                lbufs.at[p, r, pl.ds(c2 * L, L)],
                                posl[r, pl.ds(c2 * L, L)],
                            )

                @pl.loop(0, NCOORD)
                def _st_l(c):
                    ro = pl.multiple_of(c * T_BLK, T_BLK)
                    co = pl.multiple_of(c * HD_L, HD_L)
                    pltpu.async_copy(
                        lbufs.at[p, pl.ds(ro, T_BLK), :],
                        out_l.at[b, pl.ds(t0, T_BLK), pl.ds(co, HD_L)],
                        sem_sl.at[p],
                    )

                p2 = (p + 2) % NBUF
                if p < 2:
                    @pl.when(qq > 0)
                    def _drain_early():
                        wait_s(p2)

                    issue_g(p2, qq * NBUF + p + 2)
                else:
                    wait_s(p2)

                    @pl.when(qq < NQ - 1)
                    def _refill_late():
                        issue_g(p2, qq * NBUF + p + 2)

        wait_s(2)
        wait_s(3)


def kernel(tokenized_words, tokenized_bbox, lang_table, tlx_table, tly_table,
           brx_table, bry_table, w_table, h_table, box_pos_table,
           text_pos_table):
    words = (
        tokenized_words.astype(jnp.int32)
        .reshape(B, NTB, T_BLK).transpose(1, 0, 2).reshape(-1)
    )
    bbox = (
        tokenized_bbox.astype(jnp.int32)
        .reshape(B, NTB, T_BLK, NCOORD).transpose(1, 0, 3, 2).reshape(-1)
    )
    offp = jnp.repeat(
        jnp.arange(NCOORD, dtype=jnp.int32) * MAX_COORD, T_BLK
    )
    comb = jnp.concatenate(
        [tlx_table, tly_table, brx_table, bry_table, w_table, h_table], axis=0
    )
    post = text_pos_table[:T]
    posl = (
        box_pos_table[:T]
        .reshape(NTB, T_BLK, NCOORD, HD_L).transpose(0, 2, 1, 3)
        .reshape(T * NCOORD, HD_L)
    )
    out_l, out_t = _emb_kernel(words, bbox, offp, lang_table, comb, post, posl)
    return out_l, out_t


# speedup vs baseline: 1.8947x; 1.0021x over previous
import functools

import jax
import jax.numpy as jnp
from jax import lax
from jax.experimental import pallas as pl
from jax.experimental.pallas import tpu as pltpu
from jax.experimental.pallas import tpu_sc as plsc

B, T = 1024, 512
HD_T, HD_L = 768, 128
NCOORD = 6
MAX_COORD = 1001
NC, NS = 2, 16
NW = NC * NS
BPW = B // NW
T_BLK = 16
NTB = T // T_BLK
LROWS = T_BLK * NCOORD
L = 16
NBUF = 4
NQ = BPW // NBUF

_mesh = plsc.VectorSubcoreMesh(
    core_axis_name="c", subcore_axis_name="s", num_cores=NC, num_subcores=NS
)


@functools.partial(
    pl.kernel,
    out_type=(
        jax.ShapeDtypeStruct((B, T, HD_T), jnp.float32),
        jax.ShapeDtypeStruct((B, T, HD_T), jnp.float32),
    ),
    mesh=_mesh,
    scratch_types=[
        pltpu.VMEM((BPW * T_BLK,), jnp.int32),
        pltpu.VMEM((BPW * LROWS,), jnp.int32),
        pltpu.VMEM((LROWS,), jnp.int32),
        pltpu.VMEM((NBUF, T_BLK, HD_T), jnp.float32),
        pltpu.VMEM((NBUF, LROWS, HD_L), jnp.float32),
        pltpu.VMEM((T_BLK, HD_T), jnp.float32),
        pltpu.VMEM((LROWS, HD_L), jnp.float32),
        pltpu.SemaphoreType.DMA((NBUF,)),
        pltpu.SemaphoreType.DMA((NBUF,)),
        pltpu.SemaphoreType.DMA((NBUF,)),
        pltpu.SemaphoreType.DMA((NBUF,)),
        pltpu.SemaphoreType.DMA,
        pltpu.SemaphoreType.DMA,
    ],
)
def _emb_kernel(words_hbm, bbox_hbm, offp_hbm, lang_hbm, comb_hbm,
                post_hbm, posl_hbm, out_l, out_t,
                widx, lidx, offp, tbufs, lbufs, post, posl,
                sem_gt, sem_gl, sem_st, sem_sl, sem_pp, sem_pl):
    wid = lax.axis_index("s") * NC + lax.axis_index("c")
    pltpu.sync_copy(offp_hbm, offp)
    wbase = wid * (BPW * T_BLK)
    bbase = wid * (BPW * LROWS)

    def issue_g(p, i):
        pltpu.async_copy(
            lang_hbm.at[widx.at[pl.ds(i * T_BLK, T_BLK)]],
            tbufs.at[p], sem_gt.at[p],
        )
        pltpu.async_copy(
            comb_hbm.at[lidx.at[pl.ds(i * LROWS, LROWS)]],
            lbufs.at[p], sem_gl.at[p],
        )

    def wait_gt(p):
        pltpu.make_async_copy(
            lang_hbm.at[widx.at[pl.ds(0, T_BLK)]], tbufs.at[p], sem_gt.at[p]
        ).wait()

    def wait_gl(p):
        pltpu.make_async_copy(
            comb_hbm.at[lidx.at[pl.ds(0, LROWS)]], lbufs.at[p], sem_gl.at[p]
        ).wait()

    def wait_s(p):
        pltpu.make_async_copy(
            tbufs.at[p], out_t.at[0, pl.ds(0, T_BLK), :], sem_st.at[p]
        ).wait()

        pltpu.make_async_copy(
            lbufs.at[p], out_l.at[0, pl.ds(0, T_BLK), :], sem_sl.at[p]
        ).wait()

    @pl.loop(0, NTB)
    def _tb_loop(tb):
        t0 = tb * T_BLK
        cp_pp = pltpu.async_copy(
            post_hbm.at[pl.ds(t0, T_BLK), :], post, sem_pp
        )
        cp_pl = pltpu.async_copy(
            posl_hbm.at[pl.ds(t0 * NCOORD, LROWS), :], posl, sem_pl
        )
        pltpu.sync_copy(
            words_hbm.at[pl.ds(tb * (B * T_BLK) + wbase, BPW * T_BLK)], widx
        )
        pltpu.sync_copy(
            bbox_hbm.at[pl.ds(tb * (B * LROWS) + bbase, BPW * LROWS)], lidx
        )

        @pl.loop(0, BPW)
        def _mk_idx(g):
            for k in range(NCOORD):
                o = g * LROWS + k * L
                lidx[pl.ds(o, L)] = lidx[pl.ds(o, L)] + offp[pl.ds(k * L, L)]

        issue_g(0, 0)
        issue_g(1, 1)
        cp_pp.wait()
        cp_pl.wait()

        @pl.loop(0, NQ)
        def _q_loop(qq):
            for p in range(NBUF):
                i = qq * NBUF + p
                b = wid * BPW + i
                wait_gt(p)

                @pl.loop(0, T_BLK)
                def _add_t(r):
                    for c2 in range(HD_T // L):
                        plsc.addupdate(
                            tbufs.at[p, r, pl.ds(c2 * L, L)],
                            post[r, pl.ds(c2 * L, L)],
                        )

                pltpu.async_copy(
                    tbufs.at[p], out_t.at[b, pl.ds(t0, T_BLK), :], sem_st.at[p]
                )
                wait_gl(p)

                @pl.loop(0, LROWS // 8)
                def _add_l(rr):
                    for j in range(8):
                        r = rr * 8 + j
                        for c2 in range(HD_L // L):
                            plsc.addupdate(
                                lbufs.at[p, r, pl.ds(c2 * L, L)],
                                posl[r, pl.ds(c2 * L, L)],
                            )

                @pl.loop(0, NCOORD)
                def _st_l(c):
                    ro = pl.multiple_of(c * T_BLK, T_BLK)
                    co = pl.multiple_of(c * HD_L, HD_L)
                    pltpu.async_copy(
                        lbufs.at[p, pl.ds(ro, T_BLK), :],
                        out_l.at[b, pl.ds(t0, T_BLK), pl.ds(co, HD_L)],
                        sem_sl.at[p],
                    )

                p2 = (p + 2) % NBUF
                if p < 2:
                    @pl.when(qq > 0)
                    def _drain_early():
                        wait_s(p2)

                    issue_g(p2, qq * NBUF + p + 2)
                else:
                    wait_s(p2)

                    @pl.when(qq < NQ - 1)
                    def _refill_late():
                        issue_g(p2, qq * NBUF + p + 2)

        wait_s(2)
        wait_s(3)


def kernel(tokenized_words, tokenized_bbox, lang_table, tlx_table, tly_table,
           brx_table, bry_table, w_table, h_table, box_pos_table,
           text_pos_table):
    words = (
        tokenized_words.astype(jnp.int32)
        .reshape(B, NTB, T_BLK).transpose(1, 0, 2).reshape(-1)
    )
    bbox = (
        tokenized_bbox.astype(jnp.int32)
        .reshape(B, NTB, T_BLK, NCOORD).transpose(1, 0, 3, 2).reshape(-1)
    )
    offp = jnp.repeat(
        jnp.arange(NCOORD, dtype=jnp.int32) * MAX_COORD, T_BLK
    )
    comb = jnp.concatenate(
        [tlx_table, tly_table, brx_table, bry_table, w_table, h_table], axis=0
    )
    post = text_pos_table[:T]
    posl = (
        box_pos_table[:T]
        .reshape(NTB, T_BLK, NCOORD, HD_L).transpose(0, 2, 1, 3)
        .reshape(T * NCOORD, HD_L)
    )
    out_l, out_t = _emb_kernel(words, bbox, offp, lang_table, comb, post, posl)
    return out_l, out_t
